# baseline (speedup 1.0000x reference)
import functools

import jax
import numpy as np
import jax.numpy as jnp
from jax import lax
from jax.experimental import pallas as pl
from jax.experimental.pallas import tpu as pltpu
from jax.experimental.pallas import tpu_sc as plsc

VOCAB = 100000
EMBED = 64
BATCH = 4096
CTX = 20

NC = 2
NS = 16
NW = NC * NS
B_PER_W = BATCH // NW
NVREG = EMBED // 16


def _pool_sc(idx_arranged, packed_table):
    mesh = plsc.VectorSubcoreMesh(core_axis_name="c", subcore_axis_name="s")

    @functools.partial(
        pl.kernel,
        mesh=mesh,
        compiler_params=pltpu.CompilerParams(
            use_tc_tiling_on_sc=False, needs_layout_passes=False),
        out_type=jax.ShapeDtypeStruct((BATCH, EMBED), jnp.float32),
        scratch_types=[
            pltpu.VMEM((CTX, B_PER_W), jnp.int32),
            pltpu.VMEM((B_PER_W, 128), jnp.bfloat16),
            pltpu.VMEM((B_PER_W, 128), jnp.bfloat16),
            pltpu.VMEM((B_PER_W, 128), jnp.bfloat16),
            pltpu.VMEM((B_PER_W, 128), jnp.bfloat16),
            pltpu.VMEM((B_PER_W, EMBED), jnp.float32),
            pltpu.SemaphoreType.DMA,
            pltpu.SemaphoreType.DMA,
            pltpu.SemaphoreType.DMA,
            pltpu.SemaphoreType.DMA,
        ],
    )
    def k(idx_hbm, table_hbm, out_hbm, idx_v, rows0, rows1, rows2, rows3,
          acc_v, sem0, sem1, sem2, sem3):
        wid = lax.axis_index("s") * NC + lax.axis_index("c")
        base = wid * B_PER_W
        bufs = ((rows0, sem0), (rows1, sem1), (rows2, sem2), (rows3, sem3))
        pltpu.sync_copy(idx_hbm.at[wid], idx_v)
        for kk in range(4):
            pltpu.async_copy(table_hbm.at[idx_v.at[kk]], *bufs[kk])

        zero = jnp.zeros((16,), jnp.float32)

        def zero_body(i, _):
            for j in range(NVREG):
                acc_v[i, pl.ds(j * 16, 16)] = zero
            return 0

        lax.fori_loop(0, B_PER_W, zero_body, 0)

        def accum(rows_v):
            def row_body(i, _):
                r = i * 2
                for dr in range(2):
                    for m in range(2):
                        xb = rows_v[r + dr, pl.ds(32 * m, 32)]
                        lo, hi = plsc.unpack(
                            xb, format=plsc.PackFormat.INTERLEAVED)
                        sl0 = pl.ds(32 * m, 16)
                        sl1 = pl.ds(32 * m + 16, 16)
                        acc_v[r + dr, sl0] = acc_v[r + dr, sl0] + lo
                        acc_v[r + dr, sl1] = acc_v[r + dr, sl1] + hi
                return 0

            lax.fori_loop(0, B_PER_W // 2, row_body, 0)

        def ctx_body(t, _):
            c_base = 4 * t
            for kk in range(4):
                c = c_base + kk
                rv, sm = bufs[kk]
                pltpu.make_async_copy(
                    table_hbm.at[idx_v.at[c]], rv, sm).wait()
                accum(rv)

                @pl.when(c + 4 < CTX)
                def _():
                    pltpu.async_copy(table_hbm.at[idx_v.at[c + 4]], rv, sm)

            return 0

        lax.fori_loop(0, CTX // 4, ctx_body, 0)
        pltpu.sync_copy(acc_v, out_hbm.at[pl.ds(base, B_PER_W)])

    return k(idx_arranged, packed_table)


_TC = 8192


def _pack_body(x_ref, o_ref):
    x = (x_ref[...].T * jnp.float32(1.0 / CTX)).astype(jnp.bfloat16)
    o_ref[...] = jnp.concatenate(
        [x, jnp.zeros((_TC, EMBED), jnp.bfloat16)], axis=1)


def _pack_table_tc(emb_table):
    npk = pl.cdiv(VOCAB, _TC)
    return pl.pallas_call(
        _pack_body,
        grid=(npk,),
        in_specs=[pl.BlockSpec((EMBED, _TC), lambda i: (0, i))],
        out_specs=pl.BlockSpec((_TC, 128), lambda i: (i, 0)),
        out_shape=jax.ShapeDtypeStruct((VOCAB, 128), jnp.bfloat16),
    )(emb_table.T)


_BN = 1024


def _mm_body(w_ref, x_ref, o_ref):
    o_ref[...] = lax.dot_general(
        w_ref[...], x_ref[...],
        dimension_numbers=(((0,), (1,)), ((), ())),
        preferred_element_type=jnp.float32,
    )


def _project_tc(pooled, linear_w):
    nbn = pl.cdiv(VOCAB, _BN)
    out_t = pl.pallas_call(
        _mm_body,
        grid=(nbn,),
        in_specs=[
            pl.BlockSpec((EMBED, _BN), lambda i: (0, i)),
            pl.BlockSpec((BATCH, EMBED), lambda i: (0, 0)),
        ],
        out_specs=pl.BlockSpec((_BN, BATCH), lambda i: (i, 0)),
        out_shape=jax.ShapeDtypeStruct((VOCAB, BATCH), jnp.float32),
    )(linear_w.T, pooled)
    return out_t.T


_T = np.arange(EMBED)
_UNSPLIT = 32 * (_T // 32) + 16 * (_T % 2) + (_T % 32) // 2


def kernel(inputs, emb_table, linear_w):
    idx = jnp.asarray(inputs, jnp.int32)
    idx_arranged = idx.T.reshape(CTX, NW, B_PER_W).transpose(1, 0, 2)
    pooled = _pool_sc(idx_arranged, _pack_table_tc(emb_table))
    return _project_tc(pooled[:, _UNSPLIT], linear_w)


# speedup vs baseline: 1.0888x; 1.0888x over previous
import functools

import jax
import jax.numpy as jnp
from jax import lax
from jax.experimental import pallas as pl
from jax.experimental.pallas import tpu as pltpu
from jax.experimental.pallas import tpu_sc as plsc

VOCAB = 100000
EMBED = 64
BATCH = 4096
CTX = 20

NC = 2
NS = 16
NW = NC * NS
B_PER_W = BATCH // NW
NVREG = EMBED // 16


def _pool_sc(idx_arranged, packed_table):
    mesh = plsc.VectorSubcoreMesh(core_axis_name="c", subcore_axis_name="s")

    @functools.partial(
        pl.kernel,
        mesh=mesh,
        compiler_params=pltpu.CompilerParams(use_tc_tiling_on_sc=False),
        out_type=jax.ShapeDtypeStruct((BATCH, EMBED), jnp.float32),
        scratch_types=[
            pltpu.VMEM((CTX, B_PER_W), jnp.int32),
            pltpu.VMEM((B_PER_W, 128), jnp.float32),
            pltpu.VMEM((B_PER_W, 128), jnp.float32),
            pltpu.VMEM((B_PER_W, 128), jnp.float32),
            pltpu.VMEM((B_PER_W, 128), jnp.float32),
            pltpu.VMEM((B_PER_W, EMBED), jnp.float32),
            pltpu.SemaphoreType.DMA,
            pltpu.SemaphoreType.DMA,
            pltpu.SemaphoreType.DMA,
            pltpu.SemaphoreType.DMA,
        ],
    )
    def k(idx_hbm, table_hbm, out_hbm, idx_v, rows0, rows1, rows2, rows3,
          acc_v, sem0, sem1, sem2, sem3):
        wid = lax.axis_index("s") * NC + lax.axis_index("c")
        base = wid * B_PER_W
        bufs = ((rows0, sem0), (rows1, sem1), (rows2, sem2), (rows3, sem3))
        pltpu.sync_copy(idx_hbm.at[wid], idx_v)
        for kk in range(4):
            pltpu.async_copy(table_hbm.at[idx_v.at[kk]], *bufs[kk])

        zero = jnp.zeros((16,), jnp.float32)

        def zero_body(i, _):
            for j in range(NVREG):
                acc_v[i, pl.ds(j * 16, 16)] = zero
            return 0

        lax.fori_loop(0, B_PER_W, zero_body, 0)

        def accum(rows_v):
            def row_body(i, _):
                r = i * 2
                for dr in range(2):
                    for j in range(NVREG):
                        sl = pl.ds(j * 16, 16)
                        acc_v[r + dr, sl] = (
                            acc_v[r + dr, sl] + rows_v[r + dr, sl])
                return 0

            lax.fori_loop(0, B_PER_W // 2, row_body, 0)

        def ctx_body(t, _):
            c_base = 4 * t
            for kk in range(4):
                c = c_base + kk
                rv, sm = bufs[kk]
                pltpu.make_async_copy(
                    table_hbm.at[idx_v.at[c]], rv, sm).wait()
                accum(rv)

                @pl.when(c + 4 < CTX)
                def _():
                    pltpu.async_copy(table_hbm.at[idx_v.at[c + 4]], rv, sm)

            return 0

        lax.fori_loop(0, CTX // 4, ctx_body, 0)
        pltpu.sync_copy(acc_v, out_hbm.at[pl.ds(base, B_PER_W)])

    return k(idx_arranged, packed_table)


_TC = 8192


def _pack_body(x_ref, o_ref):
    x = x_ref[...].T * jnp.float32(1.0 / CTX)
    o_ref[...] = jnp.concatenate(
        [x, jnp.zeros((_TC, EMBED), jnp.float32)], axis=1)


def _pack_table_tc(emb_table):
    npk = pl.cdiv(VOCAB, _TC)
    return pl.pallas_call(
        _pack_body,
        grid=(npk,),
        in_specs=[pl.BlockSpec((EMBED, _TC), lambda i: (0, i))],
        out_specs=pl.BlockSpec((_TC, 128), lambda i: (i, 0)),
        out_shape=jax.ShapeDtypeStruct((VOCAB, 128), jnp.float32),
    )(emb_table.T)


_BN = 1280


def _mm_body(w_ref, x_ref, o_ref):
    o_ref[...] = lax.dot_general(
        w_ref[...], x_ref[...],
        dimension_numbers=(((0,), (1,)), ((), ())),
        preferred_element_type=jnp.float32,
    )


def _project_tc(pooled, linear_w):
    nbn = pl.cdiv(VOCAB, _BN)
    out_t = pl.pallas_call(
        _mm_body,
        grid=(nbn,),
        in_specs=[
            pl.BlockSpec((EMBED, _BN), lambda i: (0, i)),
            pl.BlockSpec((BATCH, EMBED), lambda i: (0, 0)),
        ],
        out_specs=pl.BlockSpec((_BN, BATCH), lambda i: (i, 0)),
        out_shape=jax.ShapeDtypeStruct((VOCAB, BATCH), jnp.float32),
    )(linear_w.T, pooled)
    return out_t.T


def kernel(inputs, emb_table, linear_w):
    idx = jnp.asarray(inputs, jnp.int32)
    idx_arranged = idx.T.reshape(CTX, NW, B_PER_W).transpose(1, 0, 2)
    pooled = _pool_sc(idx_arranged, _pack_table_tc(emb_table))
    return _project_tc(pooled, linear_w)


# speedup vs baseline: 1.0946x; 1.0054x over previous
import functools

import jax
import jax.numpy as jnp
from jax import lax
from jax.experimental import pallas as pl
from jax.experimental.pallas import tpu as pltpu
from jax.experimental.pallas import tpu_sc as plsc

VOCAB = 100000
EMBED = 64
BATCH = 4096
CTX = 20

NC = 2
NS = 16
NW = NC * NS
B_PER_W = BATCH // NW
NVREG = EMBED // 16


def _pool_sc(idx_arranged, packed_table):
    mesh = plsc.VectorSubcoreMesh(core_axis_name="c", subcore_axis_name="s")

    @functools.partial(
        pl.kernel,
        mesh=mesh,
        compiler_params=pltpu.CompilerParams(use_tc_tiling_on_sc=False),
        out_type=jax.ShapeDtypeStruct((BATCH, 128), jnp.float32),
        scratch_types=[
            pltpu.VMEM((CTX, B_PER_W), jnp.int32),
            pltpu.VMEM((B_PER_W, 128), jnp.float32),
            pltpu.VMEM((B_PER_W, 128), jnp.float32),
            pltpu.VMEM((B_PER_W, 128), jnp.float32),
            pltpu.VMEM((B_PER_W, 128), jnp.float32),
            pltpu.VMEM((B_PER_W, EMBED), jnp.float32),
            pltpu.SemaphoreType.DMA,
            pltpu.SemaphoreType.DMA,
            pltpu.SemaphoreType.DMA,
            pltpu.SemaphoreType.DMA,
        ],
    )
    def k(idx_hbm, table_hbm, out_hbm, idx_v, rows0, rows1, rows2, rows3,
          acc_v, sem0, sem1, sem2, sem3):
        wid = lax.axis_index("s") * NC + lax.axis_index("c")
        base = wid * B_PER_W
        bufs = ((rows0, sem0), (rows1, sem1), (rows2, sem2), (rows3, sem3))
        pltpu.sync_copy(idx_hbm.at[wid], idx_v)
        for kk in range(4):
            pltpu.async_copy(table_hbm.at[idx_v.at[kk]], *bufs[kk])

        zero = jnp.zeros((16,), jnp.float32)

        def zero_body(i, _):
            for j in range(NVREG):
                acc_v[i, pl.ds(j * 16, 16)] = zero
            return 0

        lax.fori_loop(0, B_PER_W, zero_body, 0)

        def accum(rows_v):
            def row_body(i, _):
                r = i * 2
                for dr in range(2):
                    for j in range(NVREG):
                        sl = pl.ds(j * 16, 16)
                        acc_v[r + dr, sl] = (
                            acc_v[r + dr, sl] + rows_v[r + dr, sl])
                return 0

            lax.fori_loop(0, B_PER_W // 2, row_body, 0)

        def ctx_body(t, _):
            c_base = 4 * t
            for kk in range(4):
                c = c_base + kk
                rv, sm = bufs[kk]
                pltpu.make_async_copy(
                    table_hbm.at[idx_v.at[c]], rv, sm).wait()
                accum(rv)

                @pl.when(c + 4 < CTX)
                def _():
                    pltpu.async_copy(table_hbm.at[idx_v.at[c + 4]], rv, sm)

            return 0

        lax.fori_loop(0, CTX // 4, ctx_body, 0)
        pltpu.sync_copy(
            acc_v, out_hbm.at[pl.ds(base, B_PER_W), pl.ds(0, EMBED)])

    return k(idx_arranged, packed_table)


_TC = 8192


def _pack_body(x_ref, o_ref):
    x = x_ref[...].T * jnp.float32(1.0 / CTX)
    o_ref[...] = jnp.concatenate(
        [x, jnp.zeros((_TC, EMBED), jnp.float32)], axis=1)


def _pack_table_tc(emb_table):
    npk = pl.cdiv(VOCAB, _TC)
    return pl.pallas_call(
        _pack_body,
        grid=(npk,),
        in_specs=[pl.BlockSpec((EMBED, _TC), lambda i: (0, i))],
        out_specs=pl.BlockSpec((_TC, 128), lambda i: (i, 0)),
        out_shape=jax.ShapeDtypeStruct((VOCAB, 128), jnp.float32),
    )(emb_table.T)


_BN = 1024


def _mm_body(w_ref, x_ref, o_ref):
    o_ref[...] = lax.dot_general(
        w_ref[...], x_ref[:, :EMBED],
        dimension_numbers=(((0,), (1,)), ((), ())),
        preferred_element_type=jnp.float32,
    )


def _project_tc(pooled, linear_w):
    nbn = pl.cdiv(VOCAB, _BN)
    out_t = pl.pallas_call(
        _mm_body,
        grid=(nbn,),
        in_specs=[
            pl.BlockSpec((EMBED, _BN), lambda i: (0, i)),
            pl.BlockSpec((BATCH, 128), lambda i: (0, 0)),
        ],
        out_specs=pl.BlockSpec((_BN, BATCH), lambda i: (i, 0)),
        out_shape=jax.ShapeDtypeStruct((VOCAB, BATCH), jnp.float32),
    )(linear_w.T, pooled)
    return out_t.T


def kernel(inputs, emb_table, linear_w):
    idx = jnp.asarray(inputs, jnp.int32)
    idx_arranged = idx.T.reshape(CTX, NW, B_PER_W).transpose(1, 0, 2)
    pooled = _pool_sc(idx_arranged, _pack_table_tc(emb_table))
    return _project_tc(pooled, linear_w)
